# Initial kernel scaffold; baseline (speedup 1.0000x reference)
import jax
import jax.numpy as jnp
from jax.experimental import pallas as pl


def kernel(block_id, target_id):
    raise NotImplementedError("write your pallas kernel here")


# speedup vs baseline: 5.6073x; 5.6073x over previous
import functools

import jax
import jax.numpy as jnp
from jax import lax
from jax.experimental import pallas as pl
from jax.experimental.pallas import tpu as pltpu
from jax.experimental.pallas import tpu_sc as plsc

B = 16
N = 4096
L = 16
CHUNKS = N // L


def _sc_body(b_hbm, tid_hbm, diff_hbm, rows_hbm, b_v, diff_v, rows_v, tid_v):
    cid = lax.axis_index("c")
    sid = lax.axis_index("s")
    wid = sid * 2 + cid
    row = lax.rem(wid, B)

    pltpu.sync_copy(b_hbm.at[row], b_v)
    pltpu.sync_copy(tid_hbm, tid_v)
    tid = tid_v[...]

    @pl.when(wid < B)
    def _rows():
        zeros_i = jnp.zeros((L,), jnp.int32)
        ones_i = jnp.ones((L,), jnp.int32)
        iota = lax.iota(jnp.int32, L)

        def zero_body(j, c):
            rows_v[pl.ds(j * L, L)] = zeros_i
            return c

        lax.fori_loop(0, CHUNKS, zero_body, 0)

        def body(j, n_vec):
            off = j * L
            v = b_v[pl.ds(off, L)]
            m = v == tid
            cum = plsc.cumsum(jnp.where(m, ones_i, zeros_i))
            pos = n_vec + cum - 1
            plsc.store_scatter(rows_v, [pos], iota + off, mask=m)
            return n_vec + plsc.all_reduce_population_count(m)

        lax.fori_loop(0, CHUNKS, body, zeros_i)
        pltpu.sync_copy(rows_v, rows_hbm.at[row])

    @pl.when(wid >= B)
    def _diff():
        onef = jnp.ones((L,), jnp.float32)
        zerof = jnp.zeros((L,), jnp.float32)

        def body(j, c):
            off = j * L
            v = b_v[pl.ds(off, L)]
            diff_v[pl.ds(off, L)] = jnp.where(v == tid, onef, zerof)
            return c

        lax.fori_loop(0, CHUNKS, body, 0)
        pltpu.sync_copy(diff_v, diff_hbm.at[row])


_sc_call = pl.kernel(
    _sc_body,
    out_type=(
        jax.ShapeDtypeStruct((B, N), jnp.float32),
        jax.ShapeDtypeStruct((B, N), jnp.int32),
    ),
    mesh=plsc.VectorSubcoreMesh(core_axis_name="c", subcore_axis_name="s"),
    scratch_types=[
        pltpu.VMEM((N,), jnp.float32),
        pltpu.VMEM((N,), jnp.float32),
        pltpu.VMEM((N,), jnp.int32),
        pltpu.VMEM((L,), jnp.float32),
    ],
    compiler_params=pltpu.CompilerParams(needs_layout_passes=False),
)


def kernel(block_id, target_id):
    b = jnp.squeeze(block_id, -1)
    tidf = jnp.asarray(target_id, jnp.float32) + 1.0
    tid_vec = jnp.broadcast_to(tidf, (L,))
    diff, rows = _sc_call(b, tid_vec)
    return diff[..., None, None], rows


# speedup vs baseline: 6.2413x; 1.1131x over previous
import functools

import jax
import jax.numpy as jnp
from jax import lax
from jax.experimental import pallas as pl
from jax.experimental.pallas import tpu as pltpu
from jax.experimental.pallas import tpu_sc as plsc

B = 16
N = 4096
L = 16
CHUNKS = N // L


def _sc_body(b_hbm, tid_hbm, diff_hbm, rows_hbm, b_v, diff_v, rows_v, tid_v):
    cid = lax.axis_index("c")
    sid = lax.axis_index("s")
    wid = sid * 2 + cid
    row = lax.rem(wid, B)

    pltpu.sync_copy(b_hbm.at[row], b_v)
    pltpu.sync_copy(tid_hbm, tid_v)
    tid = tid_v[...]

    UNROLL = 4

    @pl.when(wid < B)
    def _rows():
        zeros_i = jnp.zeros((L,), jnp.int32)
        ones_i = jnp.ones((L,), jnp.int32)
        iota = lax.iota(jnp.int32, L)

        def body(g, n_vec):
            base = g * (UNROLL * L)
            vs = [b_v[pl.ds(base + k * L, L)] for k in range(UNROLL)]
            ms = [v == tid for v in vs]
            cums = [plsc.cumsum(jnp.where(m, ones_i, zeros_i)) for m in ms]
            pcs = [plsc.all_reduce_population_count(m) for m in ms]
            n_k = n_vec
            for k in range(UNROLL):
                pos = n_k + cums[k] - 1
                plsc.store_scatter(
                    rows_v, [pos], iota + (base + k * L), mask=ms[k])
                n_k = n_k + pcs[k]
            return n_k

        n_vec = lax.fori_loop(0, CHUNKS // UNROLL, body, zeros_i)

        n = jnp.max(n_vec)
        j0 = n // L

        @pl.when(j0 < CHUNKS)
        def _boundary():
            off = j0 * L
            cur = rows_v[pl.ds(off, L)]
            rows_v[pl.ds(off, L)] = jnp.where((iota + off) < n, cur, zeros_i)

        def ztail(j, c):
            rows_v[pl.ds(j * L, L)] = zeros_i
            return c

        lax.fori_loop(j0 + 1, CHUNKS, ztail, 0)
        pltpu.sync_copy(rows_v, rows_hbm.at[row])

    @pl.when(wid >= B)
    def _diff():
        onef = jnp.ones((L,), jnp.float32)
        zerof = jnp.zeros((L,), jnp.float32)

        def body(g, c):
            base = g * (UNROLL * L)
            for k in range(UNROLL):
                off = base + k * L
                v = b_v[pl.ds(off, L)]
                diff_v[pl.ds(off, L)] = jnp.where(v == tid, onef, zerof)
            return c

        lax.fori_loop(0, CHUNKS // UNROLL, body, 0)
        pltpu.sync_copy(diff_v, diff_hbm.at[row])


_sc_call = pl.kernel(
    _sc_body,
    out_type=(
        jax.ShapeDtypeStruct((B, N), jnp.float32),
        jax.ShapeDtypeStruct((B, N), jnp.int32),
    ),
    mesh=plsc.VectorSubcoreMesh(core_axis_name="c", subcore_axis_name="s"),
    scratch_types=[
        pltpu.VMEM((N,), jnp.float32),
        pltpu.VMEM((N,), jnp.float32),
        pltpu.VMEM((N,), jnp.int32),
        pltpu.VMEM((L,), jnp.float32),
    ],
    compiler_params=pltpu.CompilerParams(needs_layout_passes=False),
)


def kernel(block_id, target_id):
    b = jnp.squeeze(block_id, -1)
    tidf = jnp.asarray(target_id, jnp.float32) + 1.0
    tid_vec = jnp.broadcast_to(tidf, (L,))
    diff, rows = _sc_call(b, tid_vec)
    return diff[..., None, None], rows


# speedup vs baseline: 6.3222x; 1.0130x over previous
import jax
import jax.numpy as jnp
from jax import lax
from jax.experimental import pallas as pl
from jax.experimental.pallas import tpu as pltpu
from jax.experimental.pallas import tpu_sc as plsc

B = 16
N = 4096
L = 16
CHUNKS = N // L
UNROLL = 4


def _sc_body(b_hbm, tid_hbm, diff_hbm, rows_hbm, b_v, diff_v, rows_v, tid_v):
    wid = lax.axis_index("s") * 2 + lax.axis_index("c")

    @pl.when(wid < B)
    def _work():
        row = wid
        pltpu.sync_copy(b_hbm.at[row], b_v)
        pltpu.sync_copy(tid_hbm, tid_v)
        tid = tid_v[...]
        zeros_i = jnp.zeros((L,), jnp.int32)
        ones_i = jnp.ones((L,), jnp.int32)
        onef = jnp.ones((L,), jnp.float32)
        zerof = jnp.zeros((L,), jnp.float32)
        iota = lax.iota(jnp.int32, L)

        def body(g, n_vec):
            base = g * (UNROLL * L)
            offs = [base + k * L for k in range(UNROLL)]
            vs = [b_v[pl.ds(o, L)] for o in offs]
            ms = [v == tid for v in vs]
            cums = [plsc.cumsum(jnp.where(m, ones_i, zeros_i)) for m in ms]
            pcs = [plsc.all_reduce_population_count(m) for m in ms]
            n_k = n_vec
            for k in range(UNROLL):
                diff_v[pl.ds(offs[k], L)] = jnp.where(ms[k], onef, zerof)
                rows_v[pl.ds(offs[k], L)] = zeros_i
                pos = n_k + cums[k] - 1
                plsc.store_scatter(rows_v, [pos], iota + offs[k], mask=ms[k])
                n_k = n_k + pcs[k]
            return n_k

        lax.fori_loop(0, CHUNKS // UNROLL, body, zeros_i)
        pltpu.sync_copy(diff_v, diff_hbm.at[row])
        pltpu.sync_copy(rows_v, rows_hbm.at[row])


_sc_call = pl.kernel(
    _sc_body,
    out_type=(
        jax.ShapeDtypeStruct((B, N), jnp.float32),
        jax.ShapeDtypeStruct((B, N), jnp.int32),
    ),
    mesh=plsc.VectorSubcoreMesh(core_axis_name="c", subcore_axis_name="s"),
    scratch_types=[
        pltpu.VMEM((N,), jnp.float32),
        pltpu.VMEM((N,), jnp.float32),
        pltpu.VMEM((N,), jnp.int32),
        pltpu.VMEM((L,), jnp.float32),
    ],
    compiler_params=pltpu.CompilerParams(needs_layout_passes=False),
)


def kernel(block_id, target_id):
    b = jnp.squeeze(block_id, -1)
    tidf = jnp.asarray(target_id, jnp.float32) + 1.0
    tid_vec = jnp.broadcast_to(tidf, (L,))
    diff, rows = _sc_call(b, tid_vec)
    return diff[..., None, None], rows


# speedup vs baseline: 6.4401x; 1.0186x over previous
import jax
import jax.numpy as jnp
from jax import lax
from jax.experimental import pallas as pl
from jax.experimental.pallas import tpu as pltpu
from jax.experimental.pallas import tpu_sc as plsc

B = 16
N = 4096
L = 16
CHUNKS = N // L
UNROLL = 4


def _sc_body(b_hbm, tid_hbm, diff_hbm, rows_hbm, b_v, diff_v, rows_v, tid_v):
    wid = lax.axis_index("s") * 2 + lax.axis_index("c")

    @pl.when(wid < B)
    def _work():
        row = wid
        pltpu.sync_copy(b_hbm.at[row], b_v)
        pltpu.sync_copy(tid_hbm, tid_v)
        tid = tid_v[...]
        zeros_i = jnp.zeros((L,), jnp.int32)
        ones_i = jnp.ones((L,), jnp.int32)
        onef = jnp.ones((L,), jnp.float32)
        zerof = jnp.zeros((L,), jnp.float32)
        iota = lax.iota(jnp.int32, L)

        def body(g, n_vec):
            base = g * (UNROLL * L)
            offs = [base + k * L for k in range(UNROLL)]
            vs = [b_v[pl.ds(o, L)] for o in offs]
            ms = [v == tid for v in vs]
            cums = [plsc.cumsum(jnp.where(m, ones_i, zeros_i)) for m in ms]
            pcs = [plsc.all_reduce_population_count(m) for m in ms]
            n_k = n_vec
            for k in range(UNROLL):
                diff_v[pl.ds(offs[k], L)] = jnp.where(ms[k], onef, zerof)
                rows_v[pl.ds(offs[k], L)] = zeros_i
                pos = n_k + cums[k] - 1
                plsc.store_scatter(rows_v, [pos], iota + offs[k], mask=ms[k])
                n_k = n_k + pcs[k]
            return n_k

        lax.fori_loop(0, CHUNKS // UNROLL, body, zeros_i)
        pltpu.sync_copy(diff_v, diff_hbm.at[row])
        pltpu.sync_copy(rows_v, rows_hbm.at[row])


_sc_call = pl.kernel(
    _sc_body,
    out_type=(
        jax.ShapeDtypeStruct((B, N), jnp.float32),
        jax.ShapeDtypeStruct((B, N), jnp.int32),
    ),
    mesh=plsc.VectorSubcoreMesh(core_axis_name="c", subcore_axis_name="s"),
    scratch_types=[
        pltpu.VMEM((N,), jnp.float32),
        pltpu.VMEM((N,), jnp.float32),
        pltpu.VMEM((N,), jnp.int32),
        pltpu.VMEM((L,), jnp.float32),
    ],
    compiler_params=pltpu.CompilerParams(
        needs_layout_passes=False, use_tc_tiling_on_sc=False),
)


def kernel(block_id, target_id):
    b = jnp.squeeze(block_id, -1)
    tidf = jnp.asarray(target_id, jnp.float32) + 1.0
    tid_vec = jnp.broadcast_to(tidf, (L,))
    diff, rows = _sc_call(b, tid_vec)
    return diff[..., None, None], rows


# speedup vs baseline: 6.8901x; 1.0699x over previous
import jax
import jax.numpy as jnp
from jax import lax
from jax.experimental import pallas as pl
from jax.experimental.pallas import tpu as pltpu
from jax.experimental.pallas import tpu_sc as plsc

B = 16
N = 4096
L = 16
CHUNKS = N // L
UNROLL = 4


def _sc_body(b_hbm, tid_hbm, diff_hbm, rows_hbm, b_v, diff_v, rows_v, tid_v):
    wid = lax.axis_index("s") + lax.axis_index("c") * 16

    @pl.when(wid < B)
    def _work():
        row = wid
        pltpu.sync_copy(b_hbm.at[row], b_v)
        pltpu.sync_copy(tid_hbm, tid_v)
        tid = tid_v[...]
        zeros_i = jnp.zeros((L,), jnp.int32)
        ones_i = jnp.ones((L,), jnp.int32)
        onef = jnp.ones((L,), jnp.float32)
        zerof = jnp.zeros((L,), jnp.float32)
        iota = lax.iota(jnp.int32, L)

        def body(g, n_vec):
            base = g * (UNROLL * L)
            offs = [base + k * L for k in range(UNROLL)]
            vs = [b_v[pl.ds(o, L)] for o in offs]
            ms = [v == tid for v in vs]
            cums = [plsc.cumsum(jnp.where(m, ones_i, zeros_i)) for m in ms]
            pcs = [plsc.all_reduce_population_count(m) for m in ms]
            n_k = n_vec
            for k in range(UNROLL):
                diff_v[pl.ds(offs[k], L)] = jnp.where(ms[k], onef, zerof)
                rows_v[pl.ds(offs[k], L)] = zeros_i
                pos = n_k + cums[k] - 1
                plsc.store_scatter(rows_v, [pos], iota + offs[k], mask=ms[k])
                n_k = n_k + pcs[k]
            return n_k

        lax.fori_loop(0, CHUNKS // UNROLL, body, zeros_i)
        pltpu.sync_copy(diff_v, diff_hbm.at[row])
        pltpu.sync_copy(rows_v, rows_hbm.at[row])


_sc_call = pl.kernel(
    _sc_body,
    out_type=(
        jax.ShapeDtypeStruct((B, N), jnp.float32),
        jax.ShapeDtypeStruct((B, N), jnp.int32),
    ),
    mesh=plsc.VectorSubcoreMesh(
        core_axis_name="c", subcore_axis_name="s", num_cores=1),
    scratch_types=[
        pltpu.VMEM((N,), jnp.float32),
        pltpu.VMEM((N,), jnp.float32),
        pltpu.VMEM((N,), jnp.int32),
        pltpu.VMEM((L,), jnp.float32),
    ],
    compiler_params=pltpu.CompilerParams(
        needs_layout_passes=False, use_tc_tiling_on_sc=False),
)


def kernel(block_id, target_id):
    b = jnp.squeeze(block_id, -1)
    tidf = jnp.asarray(target_id, jnp.float32) + 1.0
    tid_vec = jnp.broadcast_to(tidf, (L,))
    diff, rows = _sc_call(b, tid_vec)
    return diff[..., None, None], rows
